# baseline (speedup 1.0000x reference)
import jax
import jax.numpy as jnp
from jax import lax
from jax.experimental import pallas as pl
from jax.experimental.pallas import tpu as pltpu

N_CLS = 20
F_IN = 16
H1 = 5
H2 = 50
H1P = 8
H2P = 56
G1 = 4 * H1P
G2 = 4 * H2P
NCP = 24
GL = 128
TC = 256
CL = 128


def _sigmoid(x):
    return 0.5 * jnp.tanh(0.5 * x) + 0.5


def _cell(gates, c_prev, hp):
    i = _sigmoid(gates[0 * hp:1 * hp])
    f = _sigmoid(gates[1 * hp:2 * hp])
    g = jnp.tanh(gates[2 * hp:3 * hp])
    o = _sigmoid(gates[3 * hp:4 * hp])
    c_new = f * c_prev + i * g
    h_new = o * jnp.tanh(c_new)
    return h_new, c_new


def _make_body(tc, n_full, rem):
    n_chunks = n_full + (1 if rem else 0)

    def body(xs_ref, x0_ref, w1_ref, w2_ref, wfc_ref, bfc_ref,
             logits_ref, feat_ref, h1_s, c1_s, h2_s, c2_s):
        c_idx = pl.program_id(0)

        w1 = w1_ref[...]
        w2 = w2_ref[...]


        b_pad = h1_s.shape[1]
        n_ch = b_pad // CL

        def ones_row(width):
            return jnp.concatenate(
                [jnp.ones((1, width), jnp.float32),
                 jnp.zeros((7, width), jnp.float32)], axis=0)

        def layer1(xt, h1, c1, on):
            lhs1 = jnp.concatenate([xt, h1, on], axis=0)
            g1 = jnp.dot(w1, lhs1, preferred_element_type=jnp.float32)
            return _cell(g1, c1, H1P)

        @pl.when(c_idx == 0)
        def _init():
            z1 = jnp.zeros((H1P, b_pad), jnp.float32)
            h1_s[...], c1_s[...] = layer1(x0_ref[...], z1, z1, ones_row(b_pad))
            h2_s[...] = jnp.zeros_like(h2_s)
            c2_s[...] = jnp.zeros_like(c2_s)

        def step(k, carry):
            xt = xs_ref[k]
            on = ones_row(CL)
            out = []
            for j, (h1, c1, h2, c2) in enumerate(carry):
                sl = slice(j * CL, (j + 1) * CL)
                if j % 2 == 1:
                    lhs2 = jnp.concatenate([h1, h2, on], axis=0)
                    g2 = jnp.dot(w2, lhs2, preferred_element_type=jnp.float32)
                    h2n, c2n = _cell(g2, c2, H2P)
                    h1n, c1n = layer1(xt[:, sl], h1, c1, on)
                else:
                    h1n, c1n = layer1(xt[:, sl], h1, c1, on)
                    lhs2 = jnp.concatenate([h1, h2, on], axis=0)
                    g2 = jnp.dot(w2, lhs2, preferred_element_type=jnp.float32)
                    h2n, c2n = _cell(g2, c2, H2P)
                out.append((h1n, c1n, h2n, c2n))
            return tuple(out)

        def run_chunk(n_steps):
            carry = tuple(
                (h1_s[:, j * CL:(j + 1) * CL], c1_s[:, j * CL:(j + 1) * CL],
                 h2_s[:, j * CL:(j + 1) * CL], c2_s[:, j * CL:(j + 1) * CL])
                for j in range(n_ch))
            carry = lax.fori_loop(0, n_steps, step, carry,
                                  unroll=min(128, n_steps))
            for j, (h1, c1, h2, c2) in enumerate(carry):
                sl = slice(j * CL, (j + 1) * CL)
                h1_s[:, sl] = h1
                c1_s[:, sl] = c1
                h2_s[:, sl] = h2
                c2_s[:, sl] = c2

        if rem == 0:
            run_chunk(tc)
        else:
            @pl.when(c_idx < n_full)
            def _full():
                run_chunk(tc)

            @pl.when(c_idx == n_full)
            def _tail():
                run_chunk(rem)

        @pl.when(c_idx == n_chunks - 1)
        def _finalize():
            h2_last = h2_s[...]
            feat_ref[...] = h2_last
            logits_ref[...] = (jnp.dot(wfc_ref[...], h2_last,
                                       preferred_element_type=jnp.float32)
                               + bfc_ref[...])

    return body


def _repack(w_ih1, b1, w_hh1, w_l2, b2, w_fc, b_fc, b_pad):
    f32 = jnp.float32

    def blk1(g):
        wx = w_ih1[:, GL * g:GL * g + H1].T
        wh = w_hh1[:H1, GL * g:GL * g + H1].T
        blk = jnp.concatenate([wx, wh], axis=1)
        return jnp.pad(blk, ((0, H1P - H1), (0, H1P - H1)))

    w1 = jnp.concatenate([blk1(g) for g in range(4)], axis=0)
    b1v = jnp.concatenate(
        [jnp.pad(b1[0, GL * g:GL * g + H1], (0, H1P - H1)) for g in range(4)])
    w1 = jnp.concatenate(
        [w1, jnp.pad(b1v[:, None], ((0, 0), (0, 7)))], axis=1)

    def blk2(g):
        wa = w_l2[:H1, GL * g:GL * g + H2].T
        wb = w_l2[GL:GL + H2, GL * g:GL * g + H2].T
        blk = jnp.concatenate(
            [wa, jnp.zeros((H2, H1P - H1), f32), wb], axis=1)
        return jnp.pad(blk, ((0, H2P - H2), (0, H2P - H2)))

    w2 = jnp.concatenate([blk2(g) for g in range(4)], axis=0)
    b2v = jnp.concatenate(
        [jnp.pad(b2[0, GL * g:GL * g + H2], (0, H2P - H2)) for g in range(4)])
    w2 = jnp.concatenate(
        [w2, jnp.pad(b2v[:, None], ((0, 0), (0, 7)))], axis=1)

    wfc = jnp.pad(w_fc[:H2, :N_CLS].T, ((0, NCP - N_CLS), (0, H2P - H2)))
    bfcb = jnp.broadcast_to(
        jnp.pad(b_fc[0, :N_CLS], (0, NCP - N_CLS))[:, None], (NCP, b_pad))
    return w1, w2, wfc, bfcb


@jax.jit
def kernel(x, w_ih1, b1, w_hh1, w_l2, b2, w_fc, b_fc):
    x = x.astype(jnp.float32)
    B, T, F = x.shape
    assert F == F_IN
    b_pad = ((B + GL - 1) // GL) * GL

    tc = min(TC, T)
    n_full, rem = divmod(T, tc)
    n_chunks = n_full + (1 if rem else 0)
    t_pad = n_chunks * tc

    xt = jnp.transpose(x, (1, 2, 0))
    xt = jnp.pad(xt, ((0, t_pad - T + 1), (0, 0), (0, b_pad - B)))
    x0 = xt[0]
    xs = xt[1:]

    w1, w2, wfc, bfcb = _repack(w_ih1, b1, w_hh1, w_l2, b2, w_fc, b_fc, b_pad)

    def const_spec(shape):
        nd = len(shape)
        return pl.BlockSpec(shape, lambda c, _nd=nd: (0,) * _nd)

    body = _make_body(tc, n_full, rem)

    logits_pad, feat_pad = pl.pallas_call(
        body,
        out_shape=(jax.ShapeDtypeStruct((NCP, b_pad), jnp.float32),
                   jax.ShapeDtypeStruct((H2P, b_pad), jnp.float32)),
        grid=(n_chunks,),
        in_specs=[
            pl.BlockSpec((tc, F_IN, b_pad), lambda c: (c, 0, 0)),
            const_spec((F_IN, b_pad)),
            const_spec((G1, F_IN + H1P + 8)),
            const_spec((G2, H1P + H2P + 8)),
            const_spec((NCP, H2P)),
            const_spec((NCP, b_pad)),
        ],
        out_specs=(pl.BlockSpec((NCP, b_pad), lambda c: (0, 0)),
                   pl.BlockSpec((H2P, b_pad), lambda c: (0, 0))),
        scratch_shapes=[
            pltpu.VMEM((H1P, b_pad), jnp.float32),
            pltpu.VMEM((H1P, b_pad), jnp.float32),
            pltpu.VMEM((H2P, b_pad), jnp.float32),
            pltpu.VMEM((H2P, b_pad), jnp.float32),
        ],
        compiler_params=pltpu.CompilerParams(
            dimension_semantics=("arbitrary",)),
    )(xs, x0, w1, w2, wfc, bfcb)

    logits = logits_pad[:N_CLS, :B].T
    feat = feat_pad[:H2, :B].T
    return logits, {'features': feat}


# speedup vs baseline: 1.0734x; 1.0734x over previous
import jax
import jax.numpy as jnp
from jax import lax
from jax.experimental import pallas as pl
from jax.experimental.pallas import tpu as pltpu

N_CLS = 20
F_IN = 16
H1 = 5
H2 = 50
H1P = 8
H2P = 56
G1 = 4 * H1P
G2 = 4 * H2P
NCP = 24
GL = 128
TC = 256
CL = 128


def _sigmoid(x):
    return 0.5 * jnp.tanh(0.5 * x) + 0.5


def _cell(gates, c_prev, hp):
    i = _sigmoid(gates[0 * hp:1 * hp])
    f = _sigmoid(gates[1 * hp:2 * hp])
    g = jnp.tanh(gates[2 * hp:3 * hp])
    o = _sigmoid(gates[3 * hp:4 * hp])
    c_new = f * c_prev + i * g
    h_new = o * jnp.tanh(c_new)
    return h_new, c_new


def _make_body(tc, n_full, rem):
    n_chunks = n_full + (1 if rem else 0)

    def body(xs_ref, x0_ref, w1_ref, w2_ref, wfc_ref, bfc_ref,
             logits_ref, feat_ref, h1_s, c1_s, h2_s, c2_s):
        c_idx = pl.program_id(0)

        w1 = w1_ref[...]
        w2 = w2_ref[...]


        b_pad = h1_s.shape[1]
        n_ch = b_pad // CL

        def ones_row(width):
            return jnp.concatenate(
                [jnp.ones((1, width), jnp.float32),
                 jnp.zeros((7, width), jnp.float32)], axis=0)

        def layer1(xt, h1, c1, on):
            lhs1 = jnp.concatenate([xt, h1, on], axis=0)
            g1 = jnp.dot(w1, lhs1, preferred_element_type=jnp.float32)
            return _cell(g1, c1, H1P)

        @pl.when(c_idx == 0)
        def _init():
            z1 = jnp.zeros((H1P, b_pad), jnp.float32)
            h1_s[...], c1_s[...] = layer1(x0_ref[...], z1, z1, ones_row(b_pad))
            h2_s[...] = jnp.zeros_like(h2_s)
            c2_s[...] = jnp.zeros_like(c2_s)

        def step(k, carry):
            xt = xs_ref[k]
            on = ones_row(CL)
            out = []
            for j, (h1, c1, h2, c2) in enumerate(carry):
                sl = slice(j * CL, (j + 1) * CL)
                if j % 2 == 0:
                    lhs2 = jnp.concatenate([h1, h2, on], axis=0)
                    g2 = jnp.dot(w2, lhs2, preferred_element_type=jnp.float32)
                    h2n, c2n = _cell(g2, c2, H2P)
                    h1n, c1n = layer1(xt[:, sl], h1, c1, on)
                else:
                    h1n, c1n = layer1(xt[:, sl], h1, c1, on)
                    lhs2 = jnp.concatenate([h1, h2, on], axis=0)
                    g2 = jnp.dot(w2, lhs2, preferred_element_type=jnp.float32)
                    h2n, c2n = _cell(g2, c2, H2P)
                out.append((h1n, c1n, h2n, c2n))
            return tuple(out)

        def run_chunk(n_steps):
            carry = tuple(
                (h1_s[:, j * CL:(j + 1) * CL], c1_s[:, j * CL:(j + 1) * CL],
                 h2_s[:, j * CL:(j + 1) * CL], c2_s[:, j * CL:(j + 1) * CL])
                for j in range(n_ch))
            carry = lax.fori_loop(0, n_steps, step, carry,
                                  unroll=min(128, n_steps))
            for j, (h1, c1, h2, c2) in enumerate(carry):
                sl = slice(j * CL, (j + 1) * CL)
                h1_s[:, sl] = h1
                c1_s[:, sl] = c1
                h2_s[:, sl] = h2
                c2_s[:, sl] = c2

        if rem == 0:
            run_chunk(tc)
        else:
            @pl.when(c_idx < n_full)
            def _full():
                run_chunk(tc)

            @pl.when(c_idx == n_full)
            def _tail():
                run_chunk(rem)

        @pl.when(c_idx == n_chunks - 1)
        def _finalize():
            h2_last = h2_s[...]
            feat_ref[...] = h2_last
            logits_ref[...] = (jnp.dot(wfc_ref[...], h2_last,
                                       preferred_element_type=jnp.float32)
                               + bfc_ref[...])

    return body


def _repack(w_ih1, b1, w_hh1, w_l2, b2, w_fc, b_fc, b_pad):
    f32 = jnp.float32

    def blk1(g):
        wx = w_ih1[:, GL * g:GL * g + H1].T
        wh = w_hh1[:H1, GL * g:GL * g + H1].T
        blk = jnp.concatenate([wx, wh], axis=1)
        return jnp.pad(blk, ((0, H1P - H1), (0, H1P - H1)))

    w1 = jnp.concatenate([blk1(g) for g in range(4)], axis=0)
    b1v = jnp.concatenate(
        [jnp.pad(b1[0, GL * g:GL * g + H1], (0, H1P - H1)) for g in range(4)])
    w1 = jnp.concatenate(
        [w1, jnp.pad(b1v[:, None], ((0, 0), (0, 7)))], axis=1)

    def blk2(g):
        wa = w_l2[:H1, GL * g:GL * g + H2].T
        wb = w_l2[GL:GL + H2, GL * g:GL * g + H2].T
        blk = jnp.concatenate(
            [wa, jnp.zeros((H2, H1P - H1), f32), wb], axis=1)
        return jnp.pad(blk, ((0, H2P - H2), (0, H2P - H2)))

    w2 = jnp.concatenate([blk2(g) for g in range(4)], axis=0)
    b2v = jnp.concatenate(
        [jnp.pad(b2[0, GL * g:GL * g + H2], (0, H2P - H2)) for g in range(4)])
    w2 = jnp.concatenate(
        [w2, jnp.pad(b2v[:, None], ((0, 0), (0, 7)))], axis=1)

    wfc = jnp.pad(w_fc[:H2, :N_CLS].T, ((0, NCP - N_CLS), (0, H2P - H2)))
    bfcb = jnp.broadcast_to(
        jnp.pad(b_fc[0, :N_CLS], (0, NCP - N_CLS))[:, None], (NCP, b_pad))
    return w1, w2, wfc, bfcb


@jax.jit
def kernel(x, w_ih1, b1, w_hh1, w_l2, b2, w_fc, b_fc):
    x = x.astype(jnp.float32)
    B, T, F = x.shape
    assert F == F_IN
    b_pad = ((B + GL - 1) // GL) * GL

    tc = min(TC, T)
    n_full, rem = divmod(T, tc)
    n_chunks = n_full + (1 if rem else 0)
    t_pad = n_chunks * tc

    xt = jnp.transpose(x, (1, 2, 0))
    xt = jnp.pad(xt, ((0, t_pad - T + 1), (0, 0), (0, b_pad - B)))
    x0 = xt[0]
    xs = xt[1:]

    w1, w2, wfc, bfcb = _repack(w_ih1, b1, w_hh1, w_l2, b2, w_fc, b_fc, b_pad)

    def const_spec(shape):
        nd = len(shape)
        return pl.BlockSpec(shape, lambda c, _nd=nd: (0,) * _nd)

    body = _make_body(tc, n_full, rem)

    logits_pad, feat_pad = pl.pallas_call(
        body,
        out_shape=(jax.ShapeDtypeStruct((NCP, b_pad), jnp.float32),
                   jax.ShapeDtypeStruct((H2P, b_pad), jnp.float32)),
        grid=(n_chunks,),
        in_specs=[
            pl.BlockSpec((tc, F_IN, b_pad), lambda c: (c, 0, 0)),
            const_spec((F_IN, b_pad)),
            const_spec((G1, F_IN + H1P + 8)),
            const_spec((G2, H1P + H2P + 8)),
            const_spec((NCP, H2P)),
            const_spec((NCP, b_pad)),
        ],
        out_specs=(pl.BlockSpec((NCP, b_pad), lambda c: (0, 0)),
                   pl.BlockSpec((H2P, b_pad), lambda c: (0, 0))),
        scratch_shapes=[
            pltpu.VMEM((H1P, b_pad), jnp.float32),
            pltpu.VMEM((H1P, b_pad), jnp.float32),
            pltpu.VMEM((H2P, b_pad), jnp.float32),
            pltpu.VMEM((H2P, b_pad), jnp.float32),
        ],
        compiler_params=pltpu.CompilerParams(
            dimension_semantics=("arbitrary",)),
    )(xs, x0, w1, w2, wfc, bfcb)

    logits = logits_pad[:N_CLS, :B].T
    feat = feat_pad[:H2, :B].T
    return logits, {'features': feat}


# speedup vs baseline: 1.1494x; 1.0709x over previous
import jax
import jax.numpy as jnp
from jax import lax
from jax.experimental import pallas as pl
from jax.experimental.pallas import tpu as pltpu

N_CLS = 20
F_IN = 16
H1 = 5
H2 = 50
H1P = 8
H2P = 56
G1 = 4 * H1P
G2 = 4 * H2P
NCP = 24
GL = 128
TC = 256
CL = 128


def _sigmoid(x):
    return 0.5 * jnp.tanh(0.5 * x) + 0.5


def _cell(gates, c_prev, hp):
    i = _sigmoid(gates[0 * hp:1 * hp])
    f = _sigmoid(gates[1 * hp:2 * hp])
    g = jnp.tanh(gates[2 * hp:3 * hp])
    o = _sigmoid(gates[3 * hp:4 * hp])
    c_new = f * c_prev + i * g
    h_new = o * jnp.tanh(c_new)
    return h_new, c_new


def _make_body(tc, n_full, rem):
    n_chunks = n_full + (1 if rem else 0)

    def body(xs_ref, x0_ref, w1_ref, w2_ref, wfc_ref, bfc_ref,
             logits_ref, feat_ref, h1_s, c1_s, h2_s, c2_s):
        c_idx = pl.program_id(0)

        w1 = w1_ref[...]
        w2 = w2_ref[...]


        b_pad = h1_s.shape[1]
        n_ch = b_pad // CL

        def ones_row(width):
            return jnp.concatenate(
                [jnp.ones((1, width), jnp.float32),
                 jnp.zeros((7, width), jnp.float32)], axis=0)

        def layer1(xt, h1, c1, on):
            lhs1 = jnp.concatenate([xt, h1, on], axis=0)
            g1 = jnp.dot(w1, lhs1, preferred_element_type=jnp.float32)
            return _cell(g1, c1, H1P)

        @pl.when(c_idx == 0)
        def _init():
            z1 = jnp.zeros((H1P, b_pad), jnp.float32)
            h1_s[...], c1_s[...] = layer1(x0_ref[...].astype(jnp.float32),
                                          z1, z1, ones_row(b_pad))
            h2_s[...] = jnp.zeros_like(h2_s)
            c2_s[...] = jnp.zeros_like(c2_s)

        def step(k, carry):
            xt = xs_ref[k].astype(jnp.float32)
            on = ones_row(CL)
            out = []
            for j, (h1, c1, h2, c2) in enumerate(carry):
                sl = slice(j * CL, (j + 1) * CL)
                if j % 2 == 0:
                    lhs2 = jnp.concatenate([h1, h2, on], axis=0)
                    g2 = jnp.dot(w2, lhs2, preferred_element_type=jnp.float32)
                    h2n, c2n = _cell(g2, c2, H2P)
                    h1n, c1n = layer1(xt[:, sl], h1, c1, on)
                else:
                    h1n, c1n = layer1(xt[:, sl], h1, c1, on)
                    lhs2 = jnp.concatenate([h1, h2, on], axis=0)
                    g2 = jnp.dot(w2, lhs2, preferred_element_type=jnp.float32)
                    h2n, c2n = _cell(g2, c2, H2P)
                out.append((h1n, c1n, h2n, c2n))
            return tuple(out)

        def run_chunk(n_steps):
            carry = tuple(
                (h1_s[:, j * CL:(j + 1) * CL], c1_s[:, j * CL:(j + 1) * CL],
                 h2_s[:, j * CL:(j + 1) * CL], c2_s[:, j * CL:(j + 1) * CL])
                for j in range(n_ch))
            carry = lax.fori_loop(0, n_steps, step, carry,
                                  unroll=min(128, n_steps))
            for j, (h1, c1, h2, c2) in enumerate(carry):
                sl = slice(j * CL, (j + 1) * CL)
                h1_s[:, sl] = h1
                c1_s[:, sl] = c1
                h2_s[:, sl] = h2
                c2_s[:, sl] = c2

        if rem == 0:
            run_chunk(tc)
        else:
            @pl.when(c_idx < n_full)
            def _full():
                run_chunk(tc)

            @pl.when(c_idx == n_full)
            def _tail():
                run_chunk(rem)

        @pl.when(c_idx == n_chunks - 1)
        def _finalize():
            h2_last = h2_s[...]
            feat_ref[...] = h2_last
            logits_ref[...] = (jnp.dot(wfc_ref[...], h2_last,
                                       preferred_element_type=jnp.float32)
                               + bfc_ref[...])

    return body


def _repack(w_ih1, b1, w_hh1, w_l2, b2, w_fc, b_fc, b_pad):
    f32 = jnp.float32

    def blk1(g):
        wx = w_ih1[:, GL * g:GL * g + H1].T
        wh = w_hh1[:H1, GL * g:GL * g + H1].T
        blk = jnp.concatenate([wx, wh], axis=1)
        return jnp.pad(blk, ((0, H1P - H1), (0, H1P - H1)))

    w1 = jnp.concatenate([blk1(g) for g in range(4)], axis=0)
    b1v = jnp.concatenate(
        [jnp.pad(b1[0, GL * g:GL * g + H1], (0, H1P - H1)) for g in range(4)])
    w1 = jnp.concatenate(
        [w1, jnp.pad(b1v[:, None], ((0, 0), (0, 7)))], axis=1)

    def blk2(g):
        wa = w_l2[:H1, GL * g:GL * g + H2].T
        wb = w_l2[GL:GL + H2, GL * g:GL * g + H2].T
        blk = jnp.concatenate(
            [wa, jnp.zeros((H2, H1P - H1), f32), wb], axis=1)
        return jnp.pad(blk, ((0, H2P - H2), (0, H2P - H2)))

    w2 = jnp.concatenate([blk2(g) for g in range(4)], axis=0)
    b2v = jnp.concatenate(
        [jnp.pad(b2[0, GL * g:GL * g + H2], (0, H2P - H2)) for g in range(4)])
    w2 = jnp.concatenate(
        [w2, jnp.pad(b2v[:, None], ((0, 0), (0, 7)))], axis=1)

    wfc = jnp.pad(w_fc[:H2, :N_CLS].T, ((0, NCP - N_CLS), (0, H2P - H2)))
    bfcb = jnp.broadcast_to(
        jnp.pad(b_fc[0, :N_CLS], (0, NCP - N_CLS))[:, None], (NCP, b_pad))
    return w1, w2, wfc, bfcb


@jax.jit
def kernel(x, w_ih1, b1, w_hh1, w_l2, b2, w_fc, b_fc):
    x = x.astype(jnp.float32)
    B, T, F = x.shape
    assert F == F_IN
    b_pad = ((B + GL - 1) // GL) * GL

    tc = min(TC, T)
    n_full, rem = divmod(T, tc)
    n_chunks = n_full + (1 if rem else 0)
    t_pad = n_chunks * tc

    xt = jnp.transpose(x, (1, 2, 0)).astype(jnp.bfloat16)
    xt = jnp.pad(xt, ((0, t_pad - T + 1), (0, 0), (0, b_pad - B)))
    x0 = xt[0]
    xs = xt[1:]

    w1, w2, wfc, bfcb = _repack(w_ih1, b1, w_hh1, w_l2, b2, w_fc, b_fc, b_pad)

    def const_spec(shape):
        nd = len(shape)
        return pl.BlockSpec(shape, lambda c, _nd=nd: (0,) * _nd)

    body = _make_body(tc, n_full, rem)

    logits_pad, feat_pad = pl.pallas_call(
        body,
        out_shape=(jax.ShapeDtypeStruct((NCP, b_pad), jnp.float32),
                   jax.ShapeDtypeStruct((H2P, b_pad), jnp.float32)),
        grid=(n_chunks,),
        in_specs=[
            pl.BlockSpec((tc, F_IN, b_pad), lambda c: (c, 0, 0)),
            const_spec((F_IN, b_pad)),
            const_spec((G1, F_IN + H1P + 8)),
            const_spec((G2, H1P + H2P + 8)),
            const_spec((NCP, H2P)),
            const_spec((NCP, b_pad)),
        ],
        out_specs=(pl.BlockSpec((NCP, b_pad), lambda c: (0, 0)),
                   pl.BlockSpec((H2P, b_pad), lambda c: (0, 0))),
        scratch_shapes=[
            pltpu.VMEM((H1P, b_pad), jnp.float32),
            pltpu.VMEM((H1P, b_pad), jnp.float32),
            pltpu.VMEM((H2P, b_pad), jnp.float32),
            pltpu.VMEM((H2P, b_pad), jnp.float32),
        ],
        compiler_params=pltpu.CompilerParams(
            dimension_semantics=("arbitrary",)),
    )(xs, x0, w1, w2, wfc, bfcb)

    logits = logits_pad[:N_CLS, :B].T
    feat = feat_pad[:H2, :B].T
    return logits, {'features': feat}
